# baseline (speedup 1.0000x reference)
import functools

import jax
import jax.numpy as jnp
from jax import lax
from jax.experimental import pallas as pl
from jax.experimental.pallas import tpu as pltpu
from jax.experimental.pallas import tpu_sc as plsc

B = 4096
L = 50
D = 64
NC = 2
NS = 16
NW = NC * NS
BW = B // NW
LINES = 1000000 // 2


FOLD_CW = 1024
FOLD_GRID = (1000000 + FOLD_CW - 1) // FOLD_CW


def _pair_fold(table):
    t = table.T

    def body(in_ref, out_ref):
        x = in_ref[...].reshape(D, FOLD_CW // 2, 2)
        out_ref[:, 0:D] = x[:, :, 0].T
        out_ref[:, D : 2 * D] = x[:, :, 1].T

    return pl.pallas_call(
        body,
        grid=(FOLD_GRID,),
        in_specs=[pl.BlockSpec((D, FOLD_CW), lambda i: (0, i))],
        out_specs=pl.BlockSpec((FOLD_CW // 2, 2 * D), lambda i: (i, 0)),
        out_shape=jax.ShapeDtypeStruct((LINES, 2 * D), jnp.float32),
    )(t)


def kernel(user_id, items_ids, user_table, item_table):
    ut2 = _pair_fold(user_table)
    it2 = _pair_fold(item_table)
    iidx_t = items_ids.T

    mesh = plsc.VectorSubcoreMesh(
        core_axis_name="core", subcore_axis_name="subcore"
    )

    @functools.partial(
        pl.kernel,
        out_type=(
            jax.ShapeDtypeStruct((D, B), jnp.float32),
            jax.ShapeDtypeStruct((L, D, B), jnp.float32),
        ),
        mesh=mesh,
        scratch_types=[
            pltpu.VMEM((2, BW), jnp.int32),
            pltpu.VMEM((2, BW), jnp.int32),
            pltpu.VMEM((2, BW), jnp.int32),
            pltpu.VMEM((2 * BW, 2 * D), jnp.float32),
            pltpu.VMEM((2 * D, BW), jnp.float32),
            pltpu.SemaphoreType.DMA((2,)),
            pltpu.SemaphoreType.DMA((2,)),
            pltpu.SemaphoreType.DMA,
        ],
        compiler_params=pltpu.CompilerParams(
            use_tc_tiling_on_sc=True, needs_layout_passes=False
        ),
    )
    def run(ut_hbm, it_hbm, uid_hbm, iidx_hbm, uo_hbm, io_hbm,
            raw_v, lin_v, hof_v, lines_v, tb_v, gsem, ssem, isem):
        c = lax.axis_index("core")
        s = lax.axis_index("subcore")
        wid = s * NC + c
        b0 = wid * BW

        iota = lax.iota(jnp.int32, 16)

        def prep_idx(q):
            for i in range(BW // 16):
                v = raw_v.at[q][pl.ds(i * 16, 16)][...]
                lin_v.at[q][pl.ds(i * 16, 16)] = v >> 1
                hof_v.at[q][pl.ds(i * 16, 16)] = (v & 1) * D

        def fire_gather(table, q):
            pltpu.async_copy(
                table.at[lin_v.at[q]],
                lines_v.at[pl.ds(q * BW, BW)],
                gsem.at[q],
            )

        def wait_gather(table, q):
            pltpu.make_async_copy(
                table.at[lin_v.at[q]],
                lines_v.at[pl.ds(q * BW, BW)],
                gsem.at[q],
            ).wait()

        def extract(q):
            lines = lines_v.at[pl.ds(q * BW, BW)]
            tb = tb_v.at[pl.ds(q * D, D)]

            @pl.loop(0, BW // 16)
            def _(k0):
                rows = iota + k0 * 16
                cols0 = hof_v.at[q][pl.ds(k0 * 16, 16)][...]
                for f in range(D):
                    vec = plsc.load_gather(lines, [rows, cols0 + f])
                    tb.at[f][pl.ds(k0 * 16, 16)] = vec

        pltpu.sync_copy(uid_hbm.at[pl.ds(b0, BW)], raw_v.at[0])
        prep_idx(0)
        fire_gather(ut_hbm, 0)
        wait_gather(ut_hbm, 0)
        extract(0)
        pltpu.async_copy(
            tb_v.at[pl.ds(0, D)], uo_hbm.at[:, pl.ds(b0, BW)], ssem.at[0]
        )
        pltpu.make_async_copy(
            tb_v.at[pl.ds(0, D)], uo_hbm.at[:, pl.ds(b0, BW)], ssem.at[0]
        ).wait()

        def load_idx(l, q):
            pltpu.sync_copy(iidx_hbm.at[l, pl.ds(b0, BW)], raw_v.at[q])
            prep_idx(q)

        def fire_block_store(l, q):
            pltpu.async_copy(
                tb_v.at[pl.ds(q * D, D)],
                io_hbm.at[l, :, pl.ds(b0, BW)],
                ssem.at[q],
            )

        def wait_block_store(l, q):
            pltpu.make_async_copy(
                tb_v.at[pl.ds(q * D, D)],
                io_hbm.at[l, :, pl.ds(b0, BW)],
                ssem.at[q],
            ).wait()

        load_idx(0, 0)
        fire_gather(it_hbm, 0)

        @pl.loop(0, L, step=2)
        def _(l0):
            for q in (0, 1):
                l = l0 + q

                @pl.when(l + 1 < L)
                def _():
                    load_idx(l + 1, 1 - q)
                    fire_gather(it_hbm, 1 - q)

                wait_gather(it_hbm, q)

                @pl.when(l >= 2)
                def _():
                    wait_block_store(l - 2, q)

                extract(q)
                fire_block_store(l, q)

        wait_block_store(L - 2, 0)
        wait_block_store(L - 1, 1)

    user_t, item_t = run(ut2, it2, user_id, iidx_t)
    return user_t.T, jnp.transpose(item_t, (2, 0, 1))


# speedup vs baseline: 19.9144x; 19.9144x over previous
import functools

import jax
import jax.numpy as jnp
from jax import lax
from jax.experimental import pallas as pl
from jax.experimental.pallas import tpu as pltpu
from jax.experimental.pallas import tpu_sc as plsc

B = 4096
L = 50
D = 64
NC = 2
NS = 16
NW = NC * NS
BW = B // NW

FOLD_CW = 4096
FOLD_GRID = 123
HALF = FOLD_CW * FOLD_GRID


def _half_fold(table):
    t = table.T

    def body(a_ref, b_ref, out_ref):
        r = jax.lax.broadcasted_iota(jnp.int32, (D, D), 0)
        c = jax.lax.broadcasted_iota(jnp.int32, (D, D), 1)
        eye = (r == c).astype(jnp.float32)

        def tr(x):
            return jax.lax.dot_general(
                x, eye, (((0,), (0,)), ((), ())),
                preferred_element_type=jnp.float32,
                precision=jax.lax.Precision.HIGHEST,
            )

        out_ref[...] = jnp.concatenate(
            [tr(a_ref[...]), tr(b_ref[...])], axis=1
        )

    return pl.pallas_call(
        body,
        grid=(FOLD_GRID,),
        in_specs=[
            pl.BlockSpec((D, FOLD_CW), lambda i: (0, i)),
            pl.BlockSpec(
                (D, FOLD_CW),
                lambda i: (0, jnp.minimum(FOLD_GRID + i, 2 * FOLD_GRID - 2)),
            ),
        ],
        out_specs=pl.BlockSpec((FOLD_CW, 2 * D), lambda i: (i, 0)),
        out_shape=jax.ShapeDtypeStruct((HALF, 2 * D), jnp.float32),
    )(t, t)


_SC_PARAMS = pltpu.CompilerParams(
    use_tc_tiling_on_sc=True,
    needs_layout_passes=False,
    disable_bounds_checks=True,
)
_MESH = plsc.VectorSubcoreMesh(core_axis_name="core", subcore_axis_name="subcore")


def _worker_base():
    c = lax.axis_index("core")
    s = lax.axis_index("subcore")
    return (s * NC + c) * BW


def _prep_idx(raw_v, lin_v, hof_v, q):
    for i in range(BW // 16):
        v = raw_v.at[q][pl.ds(i * 16, 16)][...]
        h = (v >= HALF).astype(jnp.int32)
        lin_v.at[q][pl.ds(i * 16, 16)] = v - h * HALF
        hof_v.at[q][pl.ds(i * 16, 16)] = h * D


def _extract(lines_v, hof_v, tb_v, iota, q):
    tb = tb_v.at[pl.ds(q * D, D)]
    lines = lines_v.at[pl.ds(q * BW, BW)]

    @pl.loop(0, BW // 16)
    def _(k0):
        rows = iota + k0 * 16
        cols0 = hof_v.at[q][pl.ds(k0 * 16, 16)][...]
        for f in range(D):
            vec = plsc.load_gather(lines, [rows, cols0 + f])
            tb.at[f][pl.ds(k0 * 16, 16)] = vec


def _gather_start(table, lin_v, lines_v, gsem, q):
    pltpu.async_copy(
        table.at[lin_v.at[q]],
        lines_v.at[pl.ds(q * BW, BW)],
        gsem.at[q],
    )


def _gather_wait(table, lin_v, lines_v, gsem, q):
    pltpu.make_async_copy(
        table.at[lin_v.at[q]],
        lines_v.at[pl.ds(q * BW, BW)],
        gsem.at[q],
    ).wait()


def kernel(user_id, items_ids, user_table, item_table):
    it2 = _half_fold(item_table)
    ut2 = _half_fold(user_table)
    iidx_t = items_ids.T

    @functools.partial(
        pl.kernel,
        out_type=jax.ShapeDtypeStruct((L, D, B), jnp.float32),
        mesh=_MESH,
        scratch_types=[
            pltpu.VMEM((2, BW), jnp.int32),
            pltpu.VMEM((2, BW), jnp.int32),
            pltpu.VMEM((2, BW), jnp.int32),
            pltpu.VMEM((2 * BW, 2 * D), jnp.float32),
            pltpu.VMEM((2 * D, BW), jnp.float32),
            pltpu.SemaphoreType.DMA((2,)),
            pltpu.SemaphoreType.DMA((2,)),
        ],
        compiler_params=_SC_PARAMS,
    )
    def run_items(it_hbm, iidx_hbm, io_hbm,
                  raw_v, lin_v, hof_v, lines_v, tb_v, gsem, ssem):
        b0 = _worker_base()
        iota = lax.iota(jnp.int32, 16)

        def load_idx(l, q):
            pltpu.sync_copy(iidx_hbm.at[l, pl.ds(b0, BW)], raw_v.at[q])
            _prep_idx(raw_v, lin_v, hof_v, q)

        def block_store_start(l, q):
            pltpu.async_copy(
                tb_v.at[pl.ds(q * D, D)],
                io_hbm.at[l, :, pl.ds(b0, BW)],
                ssem.at[q],
            )

        def block_store_wait(l, q):
            pltpu.make_async_copy(
                tb_v.at[pl.ds(q * D, D)],
                io_hbm.at[l, :, pl.ds(b0, BW)],
                ssem.at[q],
            ).wait()

        load_idx(0, 0)
        _gather_start(it_hbm, lin_v, lines_v, gsem, 0)

        @pl.loop(0, L, step=2)
        def _(l0):
            for q in (0, 1):
                l = l0 + q

                @pl.when(l + 1 < L)
                def _():
                    load_idx(l + 1, 1 - q)
                    _gather_start(it_hbm, lin_v, lines_v, gsem, 1 - q)

                _gather_wait(it_hbm, lin_v, lines_v, gsem, q)

                @pl.when(l >= 2)
                def _():
                    block_store_wait(l - 2, q)

                _extract(lines_v, hof_v, tb_v, iota, q)
                block_store_start(l, q)

        block_store_wait(L - 2, 0)
        block_store_wait(L - 1, 1)

    @functools.partial(
        pl.kernel,
        out_type=jax.ShapeDtypeStruct((D, B), jnp.float32),
        mesh=_MESH,
        scratch_types=[
            pltpu.VMEM((1, BW), jnp.int32),
            pltpu.VMEM((1, BW), jnp.int32),
            pltpu.VMEM((1, BW), jnp.int32),
            pltpu.VMEM((BW, 2 * D), jnp.float32),
            pltpu.VMEM((D, BW), jnp.float32),
            pltpu.SemaphoreType.DMA((1,)),
            pltpu.SemaphoreType.DMA((1,)),
        ],
        compiler_params=_SC_PARAMS,
    )
    def run_user(ut_hbm, uid_hbm, uo_hbm,
                 raw_v, lin_v, hof_v, lines_v, tb_v, gsem, ssem):
        b0 = _worker_base()
        iota = lax.iota(jnp.int32, 16)

        pltpu.sync_copy(uid_hbm.at[pl.ds(b0, BW)], raw_v.at[0])
        _prep_idx(raw_v, lin_v, hof_v, 0)
        _gather_start(ut_hbm, lin_v, lines_v, gsem, 0)
        _gather_wait(ut_hbm, lin_v, lines_v, gsem, 0)
        _extract(lines_v, hof_v, tb_v, iota, 0)
        pltpu.sync_copy(tb_v, uo_hbm.at[:, pl.ds(b0, BW)])

    item_t = run_items(it2, iidx_t)
    user_t = run_user(ut2, user_id)
    return user_t.T, jnp.transpose(item_t, (2, 0, 1))


# speedup vs baseline: 31.1295x; 1.5632x over previous
import functools

import jax
import jax.numpy as jnp
from jax import lax
from jax.experimental import pallas as pl
from jax.experimental.pallas import tpu as pltpu
from jax.experimental.pallas import tpu_sc as plsc

B = 4096
L = 50
D = 64
NC = 2
NS = 16
NW = NC * NS
BW = B // NW

FOLD_CW = 4096
FOLD_GRID = 123
HALF = FOLD_CW * FOLD_GRID


def _half_fold(table):
    t = table.T

    def body(a_ref, b_ref, out_ref):
        r = jax.lax.broadcasted_iota(jnp.int32, (D, D), 0)
        c = jax.lax.broadcasted_iota(jnp.int32, (D, D), 1)
        eye = (r == c).astype(jnp.float32)

        def tr(x):
            return jax.lax.dot_general(
                x, eye, (((0,), (0,)), ((), ())),
                preferred_element_type=jnp.float32,
            )

        out_ref[...] = jnp.concatenate(
            [tr(a_ref[...]), tr(b_ref[...])], axis=1
        )

    return pl.pallas_call(
        body,
        grid=(FOLD_GRID,),
        in_specs=[
            pl.BlockSpec((D, FOLD_CW), lambda i: (0, i)),
            pl.BlockSpec(
                (D, FOLD_CW),
                lambda i: (0, jnp.minimum(FOLD_GRID + i, 2 * FOLD_GRID - 2)),
            ),
        ],
        out_specs=pl.BlockSpec((FOLD_CW, 2 * D), lambda i: (i, 0)),
        out_shape=jax.ShapeDtypeStruct((HALF, 2 * D), jnp.float32),
    )(t, t)


_SC_PARAMS = pltpu.CompilerParams(
    use_tc_tiling_on_sc=True,
    needs_layout_passes=False,
    disable_bounds_checks=True,
)
_MESH = plsc.VectorSubcoreMesh(core_axis_name="core", subcore_axis_name="subcore")


def _worker_base():
    c = lax.axis_index("core")
    s = lax.axis_index("subcore")
    return (s * NC + c) * BW


def _prep_idx(raw_v, lin_v, hof_v, q):
    for i in range(BW // 16):
        v = raw_v.at[q][pl.ds(i * 16, 16)][...]
        h = (v >= HALF).astype(jnp.int32)
        lin_v.at[q][pl.ds(i * 16, 16)] = v - h * HALF
        hof_v.at[q][pl.ds(i * 16, 16)] = h * D


def _extract(lines_v, hof_v, tb_v, iota, q):
    tb = tb_v.at[pl.ds(q * D, D)]
    lines = lines_v.at[pl.ds(q * BW, BW)]

    @pl.loop(0, BW // 16)
    def _(k0):
        rows = iota + k0 * 16
        cols0 = hof_v.at[q][pl.ds(k0 * 16, 16)][...]
        for f in range(D):
            vec = plsc.load_gather(lines, [rows, cols0 + f])
            tb.at[f][pl.ds(k0 * 16, 16)] = vec


def _gather_start(table, lin_v, lines_v, gsem, q):
    pltpu.async_copy(
        table.at[lin_v.at[q]],
        lines_v.at[pl.ds(q * BW, BW)],
        gsem.at[q],
    )


def _gather_wait(table, lin_v, lines_v, gsem, q):
    pltpu.make_async_copy(
        table.at[lin_v.at[q]],
        lines_v.at[pl.ds(q * BW, BW)],
        gsem.at[q],
    ).wait()


def kernel(user_id, items_ids, user_table, item_table):
    it2 = _half_fold(item_table)
    ut2 = _half_fold(user_table)
    iidx_t = items_ids.T

    @functools.partial(
        pl.kernel,
        out_type=jax.ShapeDtypeStruct((L, D, B), jnp.float32),
        mesh=_MESH,
        scratch_types=[
            pltpu.VMEM((2, BW), jnp.int32),
            pltpu.VMEM((2, BW), jnp.int32),
            pltpu.VMEM((2, BW), jnp.int32),
            pltpu.VMEM((2 * BW, 2 * D), jnp.float32),
            pltpu.VMEM((2 * D, BW), jnp.float32),
            pltpu.SemaphoreType.DMA((2,)),
            pltpu.SemaphoreType.DMA((2,)),
        ],
        compiler_params=_SC_PARAMS,
    )
    def run_items(it_hbm, iidx_hbm, io_hbm,
                  raw_v, lin_v, hof_v, lines_v, tb_v, gsem, ssem):
        b0 = _worker_base()
        iota = lax.iota(jnp.int32, 16)

        def load_idx(l, q):
            pltpu.sync_copy(iidx_hbm.at[l, pl.ds(b0, BW)], raw_v.at[q])
            _prep_idx(raw_v, lin_v, hof_v, q)

        def block_store_start(l, q):
            pltpu.async_copy(
                tb_v.at[pl.ds(q * D, D)],
                io_hbm.at[l, :, pl.ds(b0, BW)],
                ssem.at[q],
            )

        def block_store_wait(l, q):
            pltpu.make_async_copy(
                tb_v.at[pl.ds(q * D, D)],
                io_hbm.at[l, :, pl.ds(b0, BW)],
                ssem.at[q],
            ).wait()

        load_idx(0, 0)
        _gather_start(it_hbm, lin_v, lines_v, gsem, 0)

        @pl.loop(0, L, step=2)
        def _(l0):
            for q in (0, 1):
                l = l0 + q

                @pl.when(l + 1 < L)
                def _():
                    load_idx(l + 1, 1 - q)
                    _gather_start(it_hbm, lin_v, lines_v, gsem, 1 - q)

                _gather_wait(it_hbm, lin_v, lines_v, gsem, q)

                @pl.when(l >= 2)
                def _():
                    block_store_wait(l - 2, q)

                _extract(lines_v, hof_v, tb_v, iota, q)
                block_store_start(l, q)

        block_store_wait(L - 2, 0)
        block_store_wait(L - 1, 1)

    @functools.partial(
        pl.kernel,
        out_type=jax.ShapeDtypeStruct((D, B), jnp.float32),
        mesh=_MESH,
        scratch_types=[
            pltpu.VMEM((1, BW), jnp.int32),
            pltpu.VMEM((1, BW), jnp.int32),
            pltpu.VMEM((1, BW), jnp.int32),
            pltpu.VMEM((BW, 2 * D), jnp.float32),
            pltpu.VMEM((D, BW), jnp.float32),
            pltpu.SemaphoreType.DMA((1,)),
            pltpu.SemaphoreType.DMA((1,)),
        ],
        compiler_params=_SC_PARAMS,
    )
    def run_user(ut_hbm, uid_hbm, uo_hbm,
                 raw_v, lin_v, hof_v, lines_v, tb_v, gsem, ssem):
        b0 = _worker_base()
        iota = lax.iota(jnp.int32, 16)

        pltpu.sync_copy(uid_hbm.at[pl.ds(b0, BW)], raw_v.at[0])
        _prep_idx(raw_v, lin_v, hof_v, 0)
        _gather_start(ut_hbm, lin_v, lines_v, gsem, 0)
        _gather_wait(ut_hbm, lin_v, lines_v, gsem, 0)
        _extract(lines_v, hof_v, tb_v, iota, 0)
        pltpu.sync_copy(tb_v, uo_hbm.at[:, pl.ds(b0, BW)])

    item_t = run_items(it2, iidx_t)
    user_t = run_user(ut2, user_id)
    return user_t.T, jnp.transpose(item_t, (2, 0, 1))


# speedup vs baseline: 31.1490x; 1.0006x over previous
import functools

import jax
import jax.numpy as jnp
from jax import lax
from jax.experimental import pallas as pl
from jax.experimental.pallas import tpu as pltpu
from jax.experimental.pallas import tpu_sc as plsc

B = 4096
L = 50
D = 64
NC = 2
NS = 16
NW = NC * NS
BW = B // NW

FOLD_CW = 4096
FOLD_GRID = 123
HALF = FOLD_CW * FOLD_GRID


def _half_fold(table):
    t = table.T

    def body(a_ref, b_ref, out_ref):
        r = jax.lax.broadcasted_iota(jnp.int32, (D, D), 0)
        c = jax.lax.broadcasted_iota(jnp.int32, (D, D), 1)
        eye = (r == c).astype(jnp.float32)

        def tr(x):
            return jax.lax.dot_general(
                x, eye, (((0,), (0,)), ((), ())),
                preferred_element_type=jnp.float32,
            )

        out_ref[...] = jnp.concatenate(
            [tr(a_ref[...]), tr(b_ref[...])], axis=1
        )

    return pl.pallas_call(
        body,
        grid=(FOLD_GRID,),
        in_specs=[
            pl.BlockSpec((D, FOLD_CW), lambda i: (0, i)),
            pl.BlockSpec(
                (D, FOLD_CW),
                lambda i: (0, jnp.minimum(FOLD_GRID + i, 2 * FOLD_GRID - 2)),
            ),
        ],
        out_specs=pl.BlockSpec((FOLD_CW, 2 * D), lambda i: (i, 0)),
        out_shape=jax.ShapeDtypeStruct((HALF, 2 * D), jnp.float32),
    )(t, t)


_SC_PARAMS = pltpu.CompilerParams(
    use_tc_tiling_on_sc=True,
    needs_layout_passes=False,
    disable_bounds_checks=True,
)
_MESH = plsc.VectorSubcoreMesh(core_axis_name="core", subcore_axis_name="subcore")


def _worker_base():
    c = lax.axis_index("core")
    s = lax.axis_index("subcore")
    return (s * NC + c) * BW


def _prep_idx(raw_v, lin_v, hof_v, q):
    for i in range(BW // 16):
        v = raw_v.at[q][pl.ds(i * 16, 16)][...]
        h = (v >= HALF).astype(jnp.int32)
        lin_v.at[q][pl.ds(i * 16, 16)] = v - h * HALF
        hof_v.at[q][pl.ds(i * 16, 16)] = h * D


def _extract(lines_v, hof_v, tb_v, iota, q):
    tb = tb_v.at[pl.ds(q * D, D)]
    lines = lines_v.at[pl.ds(q * BW, BW)]

    @pl.loop(0, BW // 16)
    def _(k0):
        rows = iota + k0 * 16
        cols0 = hof_v.at[q][pl.ds(k0 * 16, 16)][...]
        for f in range(D):
            vec = plsc.load_gather(lines, [rows, cols0 + f])
            tb.at[f][pl.ds(k0 * 16, 16)] = vec


def _gather_start(table, lin_v, lines_v, gsem, q):
    pltpu.async_copy(
        table.at[lin_v.at[q]],
        lines_v.at[pl.ds(q * BW, BW)],
        gsem.at[q],
    )


def _gather_wait(table, lin_v, lines_v, gsem, q):
    pltpu.make_async_copy(
        table.at[lin_v.at[q]],
        lines_v.at[pl.ds(q * BW, BW)],
        gsem.at[q],
    ).wait()


def kernel(user_id, items_ids, user_table, item_table):
    iidx_t = items_ids.T

    @functools.partial(
        pl.kernel,
        out_type=jax.ShapeDtypeStruct((L, D, B), jnp.float32),
        mesh=_MESH,
        scratch_types=[
            pltpu.VMEM((2, BW), jnp.int32),
            pltpu.VMEM((2, BW), jnp.int32),
            pltpu.VMEM((2, BW), jnp.int32),
            pltpu.VMEM((2 * BW, 2 * D), jnp.float32),
            pltpu.VMEM((2 * D, BW), jnp.float32),
            pltpu.SemaphoreType.DMA((2,)),
            pltpu.SemaphoreType.DMA((2,)),
        ],
        compiler_params=_SC_PARAMS,
    )
    def run_items(it_hbm, iidx_hbm, io_hbm,
                  raw_v, lin_v, hof_v, lines_v, tb_v, gsem, ssem):
        b0 = _worker_base()
        iota = lax.iota(jnp.int32, 16)

        def load_idx(l, q):
            pltpu.sync_copy(iidx_hbm.at[l, pl.ds(b0, BW)], raw_v.at[q])
            _prep_idx(raw_v, lin_v, hof_v, q)

        def block_store_start(l, q):
            pltpu.async_copy(
                tb_v.at[pl.ds(q * D, D)],
                io_hbm.at[l, :, pl.ds(b0, BW)],
                ssem.at[q],
            )

        def block_store_wait(l, q):
            pltpu.make_async_copy(
                tb_v.at[pl.ds(q * D, D)],
                io_hbm.at[l, :, pl.ds(b0, BW)],
                ssem.at[q],
            ).wait()

        load_idx(0, 0)
        _gather_start(it_hbm, lin_v, lines_v, gsem, 0)

        @pl.loop(0, L, step=2)
        def _(l0):
            for q in (0, 1):
                l = l0 + q

                @pl.when(l + 1 < L)
                def _():
                    load_idx(l + 1, 1 - q)
                    _gather_start(it_hbm, lin_v, lines_v, gsem, 1 - q)

                _gather_wait(it_hbm, lin_v, lines_v, gsem, q)

                @pl.when(l >= 2)
                def _():
                    block_store_wait(l - 2, q)

                _extract(lines_v, hof_v, tb_v, iota, q)
                block_store_start(l, q)

        block_store_wait(L - 2, 0)
        block_store_wait(L - 1, 1)

    @functools.partial(
        pl.kernel,
        out_type=jax.ShapeDtypeStruct((D, B), jnp.float32),
        mesh=_MESH,
        scratch_types=[
            pltpu.VMEM((1, BW), jnp.int32),
            pltpu.VMEM((1, BW), jnp.int32),
            pltpu.VMEM((1, BW), jnp.int32),
            pltpu.VMEM((BW, 2 * D), jnp.float32),
            pltpu.VMEM((D, BW), jnp.float32),
            pltpu.SemaphoreType.DMA((1,)),
            pltpu.SemaphoreType.DMA((1,)),
        ],
        compiler_params=_SC_PARAMS,
    )
    def run_user(ut_hbm, uid_hbm, uo_hbm,
                 raw_v, lin_v, hof_v, lines_v, tb_v, gsem, ssem):
        b0 = _worker_base()
        iota = lax.iota(jnp.int32, 16)

        pltpu.sync_copy(uid_hbm.at[pl.ds(b0, BW)], raw_v.at[0])
        _prep_idx(raw_v, lin_v, hof_v, 0)
        _gather_start(ut_hbm, lin_v, lines_v, gsem, 0)
        _gather_wait(ut_hbm, lin_v, lines_v, gsem, 0)
        _extract(lines_v, hof_v, tb_v, iota, 0)
        pltpu.sync_copy(tb_v, uo_hbm.at[:, pl.ds(b0, BW)])

    item_t = run_items(_half_fold(item_table), iidx_t)
    user_t = run_user(_half_fold(user_table), user_id)
    return user_t.T, jnp.transpose(item_t, (2, 0, 1))
